# baseline (speedup 1.0000x reference)
import functools

import jax
import jax.numpy as jnp
from jax import lax
from jax.experimental import pallas as pl
from jax.experimental.pallas import tpu as pltpu
from jax.experimental.pallas import tpu_sc as plsc

K = 8192
M = 8192
C = 16
N = 16
NN = N * N
THRESH = 12

NUM_CORES = 2
NUM_SUBCORES = 16
NW = NUM_CORES * NUM_SUBCORES
KPW = K // NW
G = 4
NB = KPW // G
NBUF = 2


def _decoder_body(prev_hbm, flags_hbm, idx_hbm, w_hbm, b_hbm,
                  out_hbm, act_hbm,
                  idx_v, w_v, b_v, flags_v, bufs, obs, act_v,
                  semgs, semos):
    cid = lax.axis_index("c")
    sid = lax.axis_index("s")
    wid = sid * NUM_CORES + cid
    base = wid * KPW

    pltpu.sync_copy(idx_hbm.at[pl.ds(base * C, KPW * C)], idx_v)
    pltpu.sync_copy(w_hbm.at[pl.ds(base * C, KPW * C)], w_v)
    pltpu.sync_copy(b_hbm.at[pl.ds(base, KPW)], b_v.at[pl.ds(0, KPW)])
    pltpu.sync_copy(flags_hbm, flags_v)

    def start_gather(b, buf, sem):
        pltpu.async_copy(prev_hbm.at[idx_v.at[pl.ds(b * (G * C), G * C)]],
                         buf, sem)

    def wait_gather(buf, sem):
        pltpu.make_async_copy(prev_hbm.at[pl.ds(0, G * C)], buf, sem).wait()

    def start_out(ob, b, sem):
        pltpu.async_copy(ob, out_hbm.at[pl.ds(base + b * G, G)], sem)

    def wait_out(ob, sem):
        pltpu.make_async_copy(ob, out_hbm.at[pl.ds(base, G)], sem).wait()

    def splat(v, c):
        idx = jnp.full((C,), c, jnp.int32)
        return v.at[idx].get(mode="promise_in_bounds")

    def compute_batch(b, buf, ob):
        lane0 = lax.iota(jnp.int32, C) == 0
        for d in range(G):
            k = b * G + d
            idxv = idx_v[pl.ds(k * C, C)]
            fl = plsc.load_gather(flags_v, [idxv])
            m = fl > 0.5
            cnt = plsc.all_reduce_population_count(m)
            gv = cnt >= THRESH
            wf = jnp.where(jnp.logical_and(m, gv),
                           w_v[pl.ds(k * C, C)], 0.0)
            bkv = jnp.where(gv, splat(b_v[pl.ds(k, C)], 0), 0.0)
            plsc.store_scatter(act_v, [jnp.full((C,), k, jnp.int32)],
                               jnp.where(gv, 1.0, 0.0), mask=lane0)
            wcs = [splat(wf, c) for c in range(C)]
            for jj in range(N // 2):
                acc_a = bkv
                acc_b = bkv
                for c in range(C):
                    u = buf[d * C + c, pl.ds(jj * N, N)]
                    xa = plsc.bitcast(u << 16, jnp.float32)
                    xb = plsc.bitcast(u, jnp.float32)
                    acc_a = acc_a + wcs[c] * xa
                    acc_b = acc_b + wcs[c] * xb
                for acc, half in ((acc_a, 0), (acc_b, 1)):
                    au = plsc.bitcast(acc, jnp.uint32)
                    a = plsc.bitcast(au & jnp.uint32(0x7FFFFFFF),
                                     jnp.float32)
                    e = jnp.exp(-2.0 * a)
                    t = (1.0 - e) / (1.0 + e)
                    r = plsc.bitcast(
                        plsc.bitcast(t, jnp.uint32)
                        | (au & jnp.uint32(0x80000000)), jnp.float32)
                    ob[d, pl.ds(half * (NN // 2) + jj * N, N)] = r

    for u in range(NBUF):
        start_gather(u, bufs[u], semgs[u])
        start_out(obs[u], NB - NBUF + u, semos[u])

    def body(i, carry):
        for u in range(NBUF):
            b = NBUF * i + u
            wait_gather(bufs[u], semgs[u])
            wait_out(obs[u], semos[u])
            compute_batch(b, bufs[u], obs[u])
            start_gather(jnp.minimum(b + NBUF, NB - 1), bufs[u], semgs[u])
            start_out(obs[u], b, semos[u])
        return carry

    lax.fori_loop(0, NB // NBUF, body, 0)

    for u in range(NBUF):
        wait_gather(bufs[u], semgs[u])
        wait_out(obs[u], semos[u])

    pltpu.sync_copy(act_v, act_hbm.at[pl.ds(base, KPW)])


@jax.jit
def _decoder_layer_sc(prev_flat, flags_f32, idx_flat, w_flat, b):
    mesh = plsc.VectorSubcoreMesh(core_axis_name="c", subcore_axis_name="s")
    return pl.kernel(
        _decoder_body,
        out_type=(
            jax.ShapeDtypeStruct((K, NN), jnp.float32),
            jax.ShapeDtypeStruct((K,), jnp.float32),
        ),
        mesh=mesh,
        compiler_params=pltpu.CompilerParams(needs_layout_passes=False),
        scratch_types=[
            pltpu.VMEM((KPW * C,), jnp.int32),
            pltpu.VMEM((KPW * C,), jnp.float32),
            pltpu.VMEM((KPW + C,), jnp.float32),
            pltpu.VMEM((M,), jnp.float32),
            [pltpu.VMEM((G * C, NN // 2), jnp.uint32)] * NBUF,
            [pltpu.VMEM((G, NN), jnp.float32)] * NBUF,
            pltpu.VMEM((KPW,), jnp.float32),
            [pltpu.SemaphoreType.DMA] * NBUF,
            [pltpu.SemaphoreType.DMA] * NBUF,
        ],
    )(prev_flat, flags_f32, idx_flat, w_flat, b)


def kernel(prev_outputs, prev_is_active, parent_indices, w, b):
    prev_bf = prev_outputs.reshape(M, 2, NN // 2).astype(jnp.bfloat16)
    prev_pair = jnp.stack((prev_bf[:, 0, :], prev_bf[:, 1, :]), axis=-1)
    prev_flat = lax.bitcast_convert_type(prev_pair, jnp.uint32)
    flags_f32 = prev_is_active.astype(jnp.float32)
    idx_flat = parent_indices.reshape(K * C)
    w_flat = w.reshape(K * C)
    out_flat, act = _decoder_layer_sc(prev_flat, flags_f32, idx_flat,
                                      w_flat, b)
    return out_flat.reshape(K, N, N), act > 0.5


# speedup vs baseline: 1.0165x; 1.0165x over previous
import functools

import jax
import jax.numpy as jnp
from jax import lax
from jax.experimental import pallas as pl
from jax.experimental.pallas import tpu as pltpu
from jax.experimental.pallas import tpu_sc as plsc

K = 8192
M = 8192
C = 16
N = 16
NN = N * N
THRESH = 12

NUM_CORES = 2
NUM_SUBCORES = 16
NW = NUM_CORES * NUM_SUBCORES
KPW = K // NW
G = 4
NB = KPW // G
NBUF = 2


def _decoder_body(prev_hbm, flags_hbm, idx_hbm, w_hbm, b_hbm,
                  out_hbm, act_hbm,
                  idx_v, w_v, b_v, flags_v, bufs, obs, act_v,
                  semgs, semos):
    cid = lax.axis_index("c")
    sid = lax.axis_index("s")
    wid = sid * NUM_CORES + cid
    base = wid * KPW

    pltpu.sync_copy(idx_hbm.at[pl.ds(base * C, KPW * C)], idx_v)
    pltpu.sync_copy(w_hbm.at[pl.ds(base * C, KPW * C)], w_v)
    pltpu.sync_copy(b_hbm.at[pl.ds(base, KPW)], b_v.at[pl.ds(0, KPW)])
    pltpu.sync_copy(flags_hbm, flags_v)

    def start_gather(b, buf, sem):
        pltpu.async_copy(prev_hbm.at[idx_v.at[pl.ds(b * (G * C), G * C)]],
                         buf, sem)

    def wait_gather(buf, sem):
        pltpu.make_async_copy(prev_hbm.at[pl.ds(0, G * C)], buf, sem).wait()

    def start_out(ob, b, sem):
        pltpu.async_copy(ob, out_hbm.at[pl.ds(base + b * G, G)], sem)

    def wait_out(ob, sem):
        pltpu.make_async_copy(ob, out_hbm.at[pl.ds(base, G)], sem).wait()

    def splat(v, c):
        idx = jnp.full((C,), c, jnp.int32)
        return v.at[idx].get(mode="promise_in_bounds")

    def compute_batch(b, buf, ob):
        lane0 = lax.iota(jnp.int32, C) == 0
        for d in range(G):
            k = b * G + d
            idxv = idx_v[pl.ds(k * C, C)]
            fl = plsc.load_gather(flags_v, [idxv])
            m = fl > 0.5
            cnt = plsc.all_reduce_population_count(m)
            gv = cnt >= THRESH
            wf = jnp.where(jnp.logical_and(m, gv),
                           w_v[pl.ds(k * C, C)], 0.0)
            bkv = jnp.where(gv, splat(b_v[pl.ds(k, C)], 0), 0.0)
            plsc.store_scatter(act_v, [jnp.full((C,), k, jnp.int32)],
                               jnp.where(gv, 1.0, 0.0), mask=lane0)
            wcs = [splat(wf, c) for c in range(C)]
            for j in range(N):
                acc = bkv
                for c in range(C):
                    acc = acc + wcs[c] * buf[d * C + c, pl.ds(j * N, N)]
                au = plsc.bitcast(acc, jnp.uint32)
                a = plsc.bitcast(au & jnp.uint32(0x7FFFFFFF),
                                 jnp.float32)
                e = jnp.exp(-2.0 * a)
                t = (1.0 - e) / (1.0 + e)
                r = plsc.bitcast(
                    plsc.bitcast(t, jnp.uint32)
                    | (au & jnp.uint32(0x80000000)), jnp.float32)
                ob[d, pl.ds(j * N, N)] = r

    for u in range(NBUF):
        start_gather(u, bufs[u], semgs[u])
        start_out(obs[u], NB - NBUF + u, semos[u])

    def body(i, carry):
        for u in range(NBUF):
            b = NBUF * i + u
            wait_gather(bufs[u], semgs[u])
            wait_out(obs[u], semos[u])
            compute_batch(b, bufs[u], obs[u])
            start_gather(jnp.minimum(b + NBUF, NB - 1), bufs[u], semgs[u])
            start_out(obs[u], b, semos[u])
        return carry

    lax.fori_loop(0, NB // NBUF, body, 0)

    for u in range(NBUF):
        wait_gather(bufs[u], semgs[u])
        wait_out(obs[u], semos[u])

    pltpu.sync_copy(act_v, act_hbm.at[pl.ds(base, KPW)])


@jax.jit
def _decoder_layer_sc(prev_flat, flags_f32, idx_flat, w_flat, b):
    mesh = plsc.VectorSubcoreMesh(core_axis_name="c", subcore_axis_name="s")
    return pl.kernel(
        _decoder_body,
        out_type=(
            jax.ShapeDtypeStruct((K, NN), jnp.float32),
            jax.ShapeDtypeStruct((K,), jnp.float32),
        ),
        mesh=mesh,
        compiler_params=pltpu.CompilerParams(needs_layout_passes=False),
        scratch_types=[
            pltpu.VMEM((KPW * C,), jnp.int32),
            pltpu.VMEM((KPW * C,), jnp.float32),
            pltpu.VMEM((KPW + C,), jnp.float32),
            pltpu.VMEM((M,), jnp.float32),
            [pltpu.VMEM((G * C, NN), jnp.float32)] * NBUF,
            [pltpu.VMEM((G, NN), jnp.float32)] * NBUF,
            pltpu.VMEM((KPW,), jnp.float32),
            [pltpu.SemaphoreType.DMA] * NBUF,
            [pltpu.SemaphoreType.DMA] * NBUF,
        ],
    )(prev_flat, flags_f32, idx_flat, w_flat, b)


def kernel(prev_outputs, prev_is_active, parent_indices, w, b):
    prev_flat = prev_outputs.reshape(M, NN)
    flags_f32 = prev_is_active.astype(jnp.float32)
    idx_flat = parent_indices.reshape(K * C)
    w_flat = w.reshape(K * C)
    out_flat, act = _decoder_layer_sc(prev_flat, flags_f32, idx_flat,
                                      w_flat, b)
    return out_flat.reshape(K, N, N), act > 0.5
